# Initial kernel scaffold; baseline (speedup 1.0000x reference)
import jax
import jax.numpy as jnp
from jax.experimental import pallas as pl


def kernel(adj_indices, adj_values, item_content, user_w, item_w, W_content, b_content):
    raise NotImplementedError("write your pallas kernel here")


# speedup vs baseline: 1.7988x; 1.7988x over previous
import functools

import jax
import jax.numpy as jnp
from jax import lax
from jax.experimental import pallas as pl
from jax.experimental.pallas import tpu as pltpu
from jax.experimental.pallas import tpu_sc as plsc

NU = 25000
NI = 25000
N = NU + NI
D = 64
E = 800000

NC = 2
NS = 16
HALF = N // NC
ACC_ROWS = 25088
ZROWS = 392
CHUNK = 128
NCHUNKS = E // CHUNK
OUT_BLK = 200
NOUT = HALF // OUT_BLK


def _spmm_body(src_hbm, dst_hbm, val_hbm, x_hbm, zeros_hbm, out_hbm,
               acc, idx_v, dst_v, val_v, rows_v, gsem):
  c = lax.axis_index("c")
  s = lax.axis_index("s")
  core_base = c * HALF

  for b in range(4):
    pltpu.sync_copy(zeros_hbm, acc.at[pl.ds(s * 1568 + b * ZROWS, ZROWS)])
  plsc.subcore_barrier()

  @pl.loop(s, NCHUNKS, step=NS)
  def _chunk(k):
    eb = k * CHUNK
    pltpu.sync_copy(src_hbm.at[pl.ds(eb, CHUNK)], idx_v)
    pltpu.sync_copy(dst_hbm.at[pl.ds(eb, CHUNK)], dst_v)
    pltpu.sync_copy(val_hbm.at[pl.ds(eb, CHUNK)], val_v)
    pltpu.async_copy(x_hbm.at[idx_v], rows_v, gsem).wait()

    @pl.loop(0, CHUNK // 16)
    def _dloc(j):
      d16 = dst_v[pl.ds(j * 16, 16)]
      dl = d16 - core_base
      ok = (dl >= 0) & (dl < HALF)
      dst_v[pl.ds(j * 16, 16)] = jnp.where(ok, dl, HALF)

    @pl.loop(0, CHUNK // 16)
    def _scale(g):
      v16 = val_v[pl.ds(g * 16, 16)]
      for t in range(16):
        e = g * 16 + t
        vv = jnp.broadcast_to(v16[t], (16,))
        for j in range(D // 16):
          rows_v[e, pl.ds(j * 16, 16)] = rows_v[e, pl.ds(j * 16, 16)] * vv

    pltpu.sync_copy(rows_v, acc.at[dst_v], add=True)

  plsc.subcore_barrier()

  @pl.loop(s, NOUT, step=NS)
  def _out(b):
    pltpu.sync_copy(acc.at[pl.ds(b * OUT_BLK, OUT_BLK)],
                    out_hbm.at[pl.ds(core_base + b * OUT_BLK, OUT_BLK)])


_spmm = functools.partial(
    pl.kernel,
    out_type=jax.ShapeDtypeStruct((N, D), jnp.float32),
    mesh=plsc.VectorSubcoreMesh(core_axis_name="c", subcore_axis_name="s"),
    scratch_types=[
        pltpu.VMEM_SHARED((ACC_ROWS, D), jnp.float32),
        pltpu.VMEM((CHUNK,), jnp.int32),
        pltpu.VMEM((CHUNK,), jnp.int32),
        pltpu.VMEM((CHUNK,), jnp.float32),
        pltpu.VMEM((CHUNK, D), jnp.float32),
        pltpu.SemaphoreType.DMA,
    ],
    compiler_params=pltpu.CompilerParams(use_tc_tiling_on_sc=False),
)(_spmm_body)


def _item_emb_body(ic_ref, w_ref, iw_ref, b_ref, out_ref):
  proj = lax.dot_general(ic_ref[...], w_ref[...], (((1,), (1,)), ((), ())),
                         preferred_element_type=jnp.float32)
  out_ref[...] = iw_ref[...] + proj + b_ref[...]


def _item_emb(item_content, W_content, item_w, b2):
  blk = 1000
  grid = NI // blk
  return pl.pallas_call(
      _item_emb_body,
      grid=(grid,),
      in_specs=[
          pl.BlockSpec((blk, D), lambda i: (i, 0)),
          pl.BlockSpec((D, D), lambda i: (0, 0)),
          pl.BlockSpec((blk, D), lambda i: (i, 0)),
          pl.BlockSpec((1, D), lambda i: (0, 0)),
      ],
      out_specs=pl.BlockSpec((blk, D), lambda i: (i, 0)),
      out_shape=jax.ShapeDtypeStruct((NI, D), jnp.float32),
  )(item_content, W_content, item_w, b2)


def _mean4_body(a_ref, b_ref, c_ref, d_ref, out_ref):
  out_ref[...] = (a_ref[...] + b_ref[...] + c_ref[...] + d_ref[...]) * 0.25


def _mean4(a, b, c, d):
  blk = 1000
  grid = N // blk
  spec = pl.BlockSpec((blk, D), lambda i: (i, 0))
  return pl.pallas_call(
      _mean4_body,
      grid=(grid,),
      in_specs=[spec, spec, spec, spec],
      out_specs=spec,
      out_shape=jax.ShapeDtypeStruct((N, D), jnp.float32),
  )(a, b, c, d)


def kernel(adj_indices, adj_values, item_content, user_w, item_w, W_content,
           b_content):
  dst = adj_indices[0].astype(jnp.int32)
  src = adj_indices[1].astype(jnp.int32)
  val = adj_values.astype(jnp.float32)
  b2 = b_content.reshape(1, D)

  i_emb = _item_emb(item_content, W_content, item_w, b2)
  all_emb = jnp.concatenate([user_w, i_emb], axis=0)

  zeros = jnp.zeros((ZROWS, D), jnp.float32)
  x1 = _spmm(src, dst, val, all_emb, zeros)
  x2 = _spmm(src, dst, val, x1, zeros)
  x3 = _spmm(src, dst, val, x2, zeros)

  final = _mean4(all_emb, x1, x2, x3)
  return (final[:NU], final[NU:])
